# baseline (speedup 1.0000x reference)
import jax
import jax.numpy as jnp
from jax import lax
from jax.experimental import pallas as pl
from jax.experimental.pallas import tpu as pltpu
from jax.experimental.pallas import tpu_sc as plsc

SEQ = 200
BATCH = 4096
DIM = 32
VOCAB = 1000000
NC, NS = 2, 16
NW = NC * NS
BPW = BATCH // NW
NB = 8


def _sc_body(sent_hbm, table_hbm, out_hbm, idx_v, *scr):
    bufs = scr[:NB]
    sems = scr[NB:]
    wid = lax.axis_index("s") * NC + lax.axis_index("c")
    base = wid * BPW

    pltpu.sync_copy(sent_hbm.at[:, pl.ds(base, BPW)], idx_v)

    def pbody(t, c):
        for j in range(BPW // 16):
            v = idx_v[t, pl.ds(j * 16, 16)]
            idx_v[t, pl.ds(j * 16, 16)] = (
                ((v >> 10) << 10) | ((v & 255) << 2) | ((v >> 8) & 3))
        return c
    lax.fori_loop(0, SEQ, pbody, 0)

    for b in range(NB):
        pltpu.async_copy(table_hbm.at[idx_v.at[b]], bufs[b], sems[b])

    def step(k, c):
        t = NB + NB * k
        for b in range(NB):
            pltpu.make_async_copy(table_hbm.at[idx_v.at[0]], bufs[b], sems[b]).wait()
            pltpu.async_copy(table_hbm.at[idx_v.at[t + b]], bufs[b], sems[b],
                             add=True)
        return c
    lax.fori_loop(0, (SEQ - NB) // NB, step, 0)

    for b in range(NB):
        pltpu.make_async_copy(table_hbm.at[idx_v.at[0]], bufs[b], sems[b]).wait()

    inv = jnp.float32(1.0 / SEQ)

    def fbody(i, c):
        for off in (0, 16):
            vals = [buf[i, pl.ds(off, 16)] for buf in bufs]
            while len(vals) > 1:
                vals = [vals[j] + vals[j + 1] for j in range(0, len(vals) - 1, 2)] \
                    + ([vals[-1]] if len(vals) % 2 else [])
            bufs[0][i, pl.ds(off, 16)] = vals[0] * inv
        return c
    lax.fori_loop(0, BPW, fbody, 0, unroll=8)

    pltpu.sync_copy(bufs[0], out_hbm.at[pl.ds(base, BPW), :])


TBLK = 1024
SUB = TBLK // 4
TGRID = (VOCAB + TBLK - 1) // TBLK
VROWS = TGRID * SUB
VPAD = VROWS * 4


def _tc_transpose_body(x_ref, o_ref):
    x = x_ref[...]
    o_ref[...] = jnp.concatenate(
        [x[:, q * SUB:(q + 1) * SUB].T for q in range(4)], axis=1)


def _relayout_table(table):
    tableT = table.T
    tbl128 = pl.pallas_call(
        _tc_transpose_body,
        grid=(TGRID,),
        in_specs=[pl.BlockSpec((DIM, TBLK), lambda i: (0, i))],
        out_specs=pl.BlockSpec((SUB, 128), lambda i: (i, 0)),
        out_shape=jax.ShapeDtypeStruct((VROWS, 128), jnp.float32),
    )(tableT)
    return tbl128.reshape(VPAD, DIM)


def kernel(sentence, table):
    k = pl.kernel(
        _sc_body,
        out_type=jax.ShapeDtypeStruct((BATCH, DIM), jnp.float32),
        mesh=plsc.VectorSubcoreMesh(core_axis_name="c", subcore_axis_name="s"),
        compiler_params=pltpu.CompilerParams(use_tc_tiling_on_sc=False),
        scratch_types=(
            [pltpu.VMEM((SEQ, BPW), jnp.int32)]
            + [pltpu.VMEM((BPW, DIM), jnp.float32)] * NB
            + [pltpu.SemaphoreType.DMA] * NB
        ),
    )
    return k(sentence, _relayout_table(table))
